# Initial kernel scaffold; baseline (speedup 1.0000x reference)
import jax
import jax.numpy as jnp
from jax.experimental import pallas as pl


def kernel(pos, batch, c1_W0, c1_b0, c1_g0, c1_e0, c1_W1, c1_b1, c1_g1, c1_e1, c1_W2, c1_b2, c1_g2, c1_e2, c2_W0, c2_b0, c2_g0, c2_e0, l1_W0, l1_b0, l1_g0, l1_e0, h1_W0, h1_b0, h1_g0, h1_e0, h2_W0, h2_b0, h2_g0, h2_e0, fc_W, fc_b):
    raise NotImplementedError("write your pallas kernel here")


# speedup vs baseline: 4.6072x; 4.6072x over previous
import functools

import jax
import jax.numpy as jnp
from jax import lax
from jax.experimental import pallas as pl
from jax.experimental.pallas import tpu as pltpu
from jax.experimental.pallas import tpu_sc as plsc

NPTS = 8192
KNN = 20
NGR = 8
_PREC = lax.Precision.DEFAULT
_NEG = -jnp.inf


def _dot(a, b):
    return jnp.dot(a, b, preferred_element_type=jnp.float32, precision=_PREC)


def _lrelu(h):
    return jnp.where(h >= 0, h, 0.01 * h)


def _knn_body(xb_ref, xt_ref, bc_ref, br_ref, idx_ref, d_ref):
    xb = xb_ref[...]
    xt = xt_ref[...]
    sqb = jnp.sum(xb * xb, axis=1, keepdims=True)
    sqr = jnp.sum(xt * xt, axis=0, keepdims=True)
    d = sqb + sqr - 2.0 * _dot(xb, xt)
    same = bc_ref[...] == br_ref[...]
    d_ref[...] = jnp.where(same, d, jnp.inf)

    R = xb.shape[0]
    col = lax.broadcasted_iota(jnp.int32, (R, NPTS), 1)
    kio = lax.broadcasted_iota(jnp.int32, (R, KNN), 1)

    def body(t, acc):
        d = d_ref[...]
        m = jnp.min(d, axis=1, keepdims=True)
        amin = jnp.min(jnp.where(d == m, col, NPTS), axis=1, keepdims=True)
        d_ref[...] = jnp.where(col == amin, jnp.inf, d)
        return jnp.where(kio == t, amin, acc)

    idx_ref[...] = lax.fori_loop(0, KNN, body, jnp.zeros((R, KNN), jnp.int32))


def _knn(x, bcol, brow):
    n, c = x.shape
    r = 256
    return pl.pallas_call(
        _knn_body,
        grid=(n // r,),
        in_specs=[
            pl.BlockSpec((r, c), lambda i: (i, 0)),
            pl.BlockSpec((c, n), lambda i: (0, 0)),
            pl.BlockSpec((r, 1), lambda i: (i, 0)),
            pl.BlockSpec((1, n), lambda i: (0, 0)),
        ],
        out_specs=pl.BlockSpec((r, KNN), lambda i: (i, 0)),
        out_shape=jax.ShapeDtypeStruct((n, KNN), jnp.int32),
        scratch_shapes=[pltpu.VMEM((r, NPTS), jnp.float32)],
    )(x, x.T, bcol, brow)


def _gather_rows(table, idx):
    nrow, d = table.shape
    b = idx.shape[0]
    nw = 32
    bw = b // nw
    chunk = 512 if d > 64 else 1024
    nch = bw // chunk
    mesh = plsc.VectorSubcoreMesh(core_axis_name="c", subcore_axis_name="s")

    @functools.partial(
        pl.kernel,
        out_type=jax.ShapeDtypeStruct((b, d), jnp.float32),
        mesh=mesh,
        scratch_types=[
            pltpu.VMEM((chunk,), jnp.int32),
            pltpu.VMEM((chunk, d), jnp.float32),
            pltpu.SemaphoreType.DMA,
        ],
    )
    def gk(table_hbm, idx_hbm, out_hbm, idx_v, rows_v, sem):
        wid = lax.axis_index("s") * 2 + lax.axis_index("c")

        def body(j, carry):
            base = wid * bw + j * chunk
            pltpu.sync_copy(idx_hbm.at[pl.ds(base, chunk)], idx_v)
            pltpu.async_copy(table_hbm.at[idx_v], rows_v, sem).wait()
            pltpu.sync_copy(rows_v, out_hbm.at[pl.ds(base, chunk)])
            return carry

        lax.fori_loop(0, nch, body, 0)

    return gk(table, idx)


def _edge1_body(xj_ref, xi_ref, w_ref, b_ref, h_ref):
    ci = xi_ref.shape[1]
    xi = xi_ref[...]
    el = xj_ref[0][:, :ci] - xi
    e = jnp.concatenate([xi, el], axis=1)
    h_ref[0] = _lrelu(_dot(e, w_ref[...]) + b_ref[...])


def _edge1(xj3, xi, w, brow):
    bp = 2048
    cw = xj3.shape[2]
    cout = w.shape[1]
    return pl.pallas_call(
        _edge1_body,
        grid=(KNN, NPTS // bp),
        in_specs=[
            pl.BlockSpec((1, bp, cw), lambda k, i: (k, i, 0)),
            pl.BlockSpec((bp, xi.shape[1]), lambda k, i: (i, 0)),
            pl.BlockSpec(w.shape, lambda k, i: (0, 0)),
            pl.BlockSpec((1, cout), lambda k, i: (0, 0)),
        ],
        out_specs=pl.BlockSpec((1, bp, cout), lambda k, i: (k, i, 0)),
        out_shape=jax.ShapeDtypeStruct((KNN, NPTS, cout), jnp.float32),
    )(xj3, xi, w, brow)


def _bn(x, g, e, m, den):
    return g * (x - m) / den + e


def _mlpaff_body(x_ref, g_ref, e_ref, m_ref, dn_ref, w_ref, b_ref, h_ref):
    z = _bn(x_ref[...], g_ref[...], e_ref[...], m_ref[...], dn_ref[...])
    h_ref[...] = _lrelu(_dot(z, w_ref[...]) + b_ref[...])


def _mlpaff(x, g, e, mm, den, w, brow, blk=4096):
    m, cin = x.shape
    cout = w.shape[1]
    return pl.pallas_call(
        _mlpaff_body,
        grid=(m // blk,),
        in_specs=[
            pl.BlockSpec((blk, cin), lambda i: (i, 0)),
            pl.BlockSpec((1, cin), lambda i: (0, 0)),
            pl.BlockSpec((1, cin), lambda i: (0, 0)),
            pl.BlockSpec((1, cin), lambda i: (0, 0)),
            pl.BlockSpec((1, cin), lambda i: (0, 0)),
            pl.BlockSpec((cin, cout), lambda i: (0, 0)),
            pl.BlockSpec((1, cout), lambda i: (0, 0)),
        ],
        out_specs=pl.BlockSpec((blk, cout), lambda i: (i, 0)),
        out_shape=jax.ShapeDtypeStruct((m, cout), jnp.float32),
    )(x, g, e, mm, den, w, brow)


def _mlpaffmax_body(x_ref, g_ref, e_ref, m_ref, dn_ref, w_ref, b_ref,
                    mx_ref, h_ref):
    z = _bn(x_ref[0], g_ref[...], e_ref[...], m_ref[...], dn_ref[...])
    h = _lrelu(_dot(z, w_ref[...]) + b_ref[...])
    h_ref[0] = h
    k = pl.program_id(1)

    @pl.when(k == 0)
    def _():
        mx_ref[...] = h

    @pl.when(k > 0)
    def _():
        mx_ref[...] = jnp.maximum(mx_ref[...], h)


def _mlpaffmax(h3, g, e, mm, den, w, brow):
    _, n, cin = h3.shape
    cout = w.shape[1]
    bp = 1024
    return pl.pallas_call(
        _mlpaffmax_body,
        grid=(n // bp, KNN),
        in_specs=[
            pl.BlockSpec((1, bp, cin), lambda i, k: (k, i, 0)),
            pl.BlockSpec((1, cin), lambda i, k: (0, 0)),
            pl.BlockSpec((1, cin), lambda i, k: (0, 0)),
            pl.BlockSpec((1, cin), lambda i, k: (0, 0)),
            pl.BlockSpec((1, cin), lambda i, k: (0, 0)),
            pl.BlockSpec((cin, cout), lambda i, k: (0, 0)),
            pl.BlockSpec((1, cout), lambda i, k: (0, 0)),
        ],
        out_specs=[
            pl.BlockSpec((bp, cout), lambda i, k: (i, 0)),
            pl.BlockSpec((1, bp, cout), lambda i, k: (k, i, 0)),
        ],
        out_shape=[
            jax.ShapeDtypeStruct((n, cout), jnp.float32),
            jax.ShapeDtypeStruct((KNN, n, cout), jnp.float32),
        ],
    )(h3, g, e, mm, den, w, brow)


def _edge2_body(xj_ref, xi_ref, w_ref, b_ref, mx_ref, h_ref):
    ci = xi_ref.shape[1]
    xi = xi_ref[...]
    el = xj_ref[0][:, :ci] - xi
    e = jnp.concatenate([xi, el], axis=1)
    h = _lrelu(_dot(e, w_ref[...]) + b_ref[...])
    h_ref[0] = h
    k = pl.program_id(1)

    @pl.when(k == 0)
    def _():
        mx_ref[...] = h

    @pl.when(k > 0)
    def _():
        mx_ref[...] = jnp.maximum(mx_ref[...], h)


def _edge2(xj3, xi, w, brow):
    bp = 1024
    cw = xj3.shape[2]
    cout = w.shape[1]
    return pl.pallas_call(
        _edge2_body,
        grid=(NPTS // bp, KNN),
        in_specs=[
            pl.BlockSpec((1, bp, cw), lambda i, k: (k, i, 0)),
            pl.BlockSpec((bp, xi.shape[1]), lambda i, k: (i, 0)),
            pl.BlockSpec(w.shape, lambda i, k: (0, 0)),
            pl.BlockSpec((1, cout), lambda i, k: (0, 0)),
        ],
        out_specs=[
            pl.BlockSpec((bp, cout), lambda i, k: (i, 0)),
            pl.BlockSpec((1, bp, cout), lambda i, k: (k, i, 0)),
        ],
        out_shape=[
            jax.ShapeDtypeStruct((NPTS, cout), jnp.float32),
            jax.ShapeDtypeStruct((KNN, NPTS, cout), jnp.float32),
        ],
    )(xj3, xi, w, brow)


def _aff_body(x_ref, g_ref, e_ref, m_ref, dn_ref, o_ref):
    o_ref[...] = _bn(x_ref[...], g_ref[...], e_ref[...], m_ref[...],
                     dn_ref[...])


def _affine(x, g, e, mm, den):
    m, d = x.shape
    blk = 1024
    return pl.pallas_call(
        _aff_body,
        grid=(m // blk,),
        in_specs=[
            pl.BlockSpec((blk, d), lambda i: (i, 0)),
            pl.BlockSpec((1, d), lambda i: (0, 0)),
            pl.BlockSpec((1, d), lambda i: (0, 0)),
            pl.BlockSpec((1, d), lambda i: (0, 0)),
            pl.BlockSpec((1, d), lambda i: (0, 0)),
        ],
        out_specs=pl.BlockSpec((blk, d), lambda i: (i, 0)),
        out_shape=jax.ShapeDtypeStruct((m, d), jnp.float32),
    )(x, g, e, mm, den)


def _l1_body(x1_ref, x2_ref, g_ref, e_ref, m_ref, dn_ref, w_ref, b_ref,
             bt_ref, emb_ref, st_ref):
    x2 = _bn(x2_ref[...], g_ref[...], e_ref[...], m_ref[...], dn_ref[...])
    cat = jnp.concatenate([x1_ref[...], x2], axis=1)
    h = _lrelu(_dot(cat, w_ref[...]) + b_ref[...])

    @pl.when(pl.program_id(0) == 0)
    def _():
        emb_ref[...] = jnp.full_like(emb_ref, _NEG)
        st_ref[...] = jnp.zeros_like(st_ref)

    st_ref[...] += jnp.stack(
        [jnp.sum(h, axis=0), jnp.sum(h * h, axis=0)])

    bt = bt_ref[...]
    for g in range(NGR):
        red = jnp.max(jnp.where(bt == g, h, _NEG), axis=0, keepdims=True)
        emb_ref[pl.ds(g, 1), :] = jnp.maximum(emb_ref[pl.ds(g, 1), :], red)


def _l1_segmax(x1, x2pre, g, e, mm, den, w, brow, bcol):
    n = x1.shape[0]
    cout = w.shape[1]
    bp = 1024
    return pl.pallas_call(
        _l1_body,
        grid=(n // bp,),
        in_specs=[
            pl.BlockSpec((bp, x1.shape[1]), lambda i: (i, 0)),
            pl.BlockSpec((bp, x2pre.shape[1]), lambda i: (i, 0)),
            pl.BlockSpec((1, x2pre.shape[1]), lambda i: (0, 0)),
            pl.BlockSpec((1, x2pre.shape[1]), lambda i: (0, 0)),
            pl.BlockSpec((1, x2pre.shape[1]), lambda i: (0, 0)),
            pl.BlockSpec((1, x2pre.shape[1]), lambda i: (0, 0)),
            pl.BlockSpec(w.shape, lambda i: (0, 0)),
            pl.BlockSpec((1, cout), lambda i: (0, 0)),
            pl.BlockSpec((bp, 1), lambda i: (i, 0)),
        ],
        out_specs=[
            pl.BlockSpec((NGR, cout), lambda i: (0, 0)),
            pl.BlockSpec((2, cout), lambda i: (0, 0)),
        ],
        out_shape=[
            jax.ShapeDtypeStruct((NGR, cout), jnp.float32),
            jax.ShapeDtypeStruct((2, cout), jnp.float32),
        ],
    )(x1, x2pre, g, e, mm, den, w, brow, bcol)


def _head_body(ep_ref, g5_ref, e5_ref, m5_ref, d5_ref,
               w1_ref, b1_ref, g1_ref, e1_ref,
               w2_ref, b2_ref, g2_ref, e2_ref, wf_ref, bf_ref,
               emb_ref, log_ref):
    emb = _bn(ep_ref[...], g5_ref[...], e5_ref[...], m5_ref[...], d5_ref[...])
    emb_ref[...] = emb

    def layer(x, w, b, g, e):
        h = _lrelu(_dot(x, w[...]) + b[...])
        m = jnp.mean(h, axis=0, keepdims=True)
        v = jnp.mean((h - m) * (h - m), axis=0, keepdims=True)
        return g[...] * (h - m) / jnp.sqrt(v + 1e-5) + e[...]

    h = layer(emb, w1_ref, b1_ref, g1_ref, e1_ref)
    h = layer(h, w2_ref, b2_ref, g2_ref, e2_ref)
    log_ref[...] = _dot(h, wf_ref[...]) + bf_ref[...]


def _head(embpre, g5, e5, m5, d5, w1, b1, g1, e1, w2, b2, g2, e2, wf, bf):
    return pl.pallas_call(
        _head_body,
        out_shape=[
            jax.ShapeDtypeStruct((NGR, embpre.shape[1]), jnp.float32),
            jax.ShapeDtypeStruct((NGR, wf.shape[1]), jnp.float32),
        ],
    )(embpre, g5, e5, m5, d5, w1, b1, g1, e1, w2, b2, g2, e2, wf, bf)


def _bn_params(stats, cnt):
    s, s2 = stats[0], stats[1]
    m = s / cnt
    v = s2 / cnt - m * m
    return m.reshape(1, -1), jnp.sqrt(v + 1e-5).reshape(1, -1)


def _bn_params_of(hkm):
    c = hkm.shape[2]
    hn = lax.optimization_barrier(
        hkm.transpose(1, 0, 2).reshape(NPTS * KNN, c))
    m = jnp.mean(hn, axis=0)
    v = jnp.var(hn, axis=0)
    return m.reshape(1, -1), jnp.sqrt(v + 1e-5).reshape(1, -1)


def _row(v):
    return v.reshape(1, -1)


def kernel(pos, batch, c1_W0, c1_b0, c1_g0, c1_e0, c1_W1, c1_b1, c1_g1,
           c1_e1, c1_W2, c1_b2, c1_g2, c1_e2, c2_W0, c2_b0, c2_g0, c2_e0,
           l1_W0, l1_b0, l1_g0, l1_e0, h1_W0, h1_b0, h1_g0, h1_e0,
           h2_W0, h2_b0, h2_g0, h2_e0, fc_W, fc_b):
    ne = NPTS * KNN
    bcol = batch.reshape(NPTS, 1)
    brow = batch.reshape(1, NPTS)

    posp = jnp.pad(pos, ((0, 0), (0, 5)))
    posw = jnp.pad(pos, ((0, 0), (0, 125)))
    idx1 = _knn(posp, bcol, brow)
    idx1f = idx1.T.reshape(ne)

    xj1 = _gather_rows(posw, idx1f)
    w16 = jnp.concatenate([jnp.pad(c1_W0[:3], ((0, 5), (0, 0))),
                           jnp.pad(c1_W0[3:], ((0, 5), (0, 0)))], axis=0)
    h1f = _edge1(xj1.reshape(KNN, NPTS, 128), posp, w16, _row(c1_b0))

    m1, d1 = _bn_params_of(h1f)
    h2f = _mlpaff(h1f.reshape(ne, 64), _row(c1_g0), _row(c1_e0),
                  m1, d1, c1_W1, _row(c1_b1))

    m2, d2 = _bn_params_of(h2f.reshape(KNN, NPTS, 64))
    x1pre, h3f = _mlpaffmax(h2f.reshape(KNN, NPTS, 64), _row(c1_g1),
                            _row(c1_e1), m2, d2, c1_W2, _row(c1_b2))

    m3, d3 = _bn_params_of(h3f)
    x1 = _affine(x1pre, _row(c1_g2), _row(c1_e2), m3, d3)

    idx2 = _knn(x1, bcol, brow)
    idx2f = idx2.T.reshape(ne)

    x1w = jnp.pad(x1, ((0, 0), (0, 64)))
    xj2 = _gather_rows(x1w, idx2f)
    x2pre, h4f = _edge2(xj2.reshape(KNN, NPTS, 128), x1, c2_W0, _row(c2_b0))

    m4, d4 = _bn_params_of(h4f)

    embpre, st5 = _l1_segmax(x1, x2pre, _row(c2_g0), _row(c2_e0), m4, d4,
                             l1_W0, _row(l1_b0), bcol)

    m5, d5 = _bn_params(st5, float(NPTS))

    emb, logits = _head(
        embpre, _row(l1_g0), _row(l1_e0), m5, d5,
        h1_W0, _row(h1_b0), _row(h1_g0), _row(h1_e0),
        h2_W0, _row(h2_b0), _row(h2_g0), _row(h2_e0),
        fc_W, _row(fc_b))
    return (logits, emb)


# speedup vs baseline: 5.0338x; 1.0926x over previous
import functools

import jax
import jax.numpy as jnp
from jax import lax
from jax.experimental import pallas as pl
from jax.experimental.pallas import tpu as pltpu
from jax.experimental.pallas import tpu_sc as plsc

NPTS = 8192
KNN = 20
NGR = 8
_PREC = lax.Precision.DEFAULT
_NEG = -jnp.inf


def _dot(a, b):
    return jnp.dot(a, b, preferred_element_type=jnp.float32, precision=_PREC)


def _lrelu(h):
    return jnp.where(h >= 0, h, 0.01 * h)


_KR = 256
_KW = 512
_KC = NPTS // _KW


def _knn_body(bs_ref, xb_ref, xt_ref, bc_ref, br_ref, idx_ref, d_ref, m_ref):
    i = pl.program_id(0)
    xb = xb_ref[...]
    sqb = jnp.sum(xb * xb, axis=1, keepdims=True)
    g_lo = bs_ref[0, i * _KR]
    g_hi = bs_ref[0, i * _KR + _KR - 1]

    idx_ref[...] = jnp.zeros((_KR, KNN), jnp.int32)

    active = []
    for c in range(_KC):
        cs = c * _KW
        act = jnp.logical_and(bs_ref[0, cs + _KW - 1] >= g_lo,
                              bs_ref[0, cs] <= g_hi)
        active.append(act)

        @pl.when(act)
        def _(c=c, cs=cs):
            xt = xt_ref[:, cs:cs + _KW]
            sqr = jnp.sum(xt * xt, axis=0, keepdims=True)
            d = sqb + sqr - 2.0 * _dot(xb, xt)
            d = jnp.where(bc_ref[...] == br_ref[:, cs:cs + _KW], d, jnp.inf)
            d_ref[:, cs:cs + _KW] = d
            m_ref[:, c:c + 1] = jnp.min(d, axis=1, keepdims=True)

        @pl.when(jnp.logical_not(act))
        def _(c=c):
            m_ref[:, c:c + 1] = jnp.full((_KR, 1), jnp.inf)

    cio = lax.broadcasted_iota(jnp.int32, (_KR, _KC), 1)
    kio = lax.broadcasted_iota(jnp.int32, (_KR, KNN), 1)
    wio = lax.broadcasted_iota(jnp.int32, (_KR, _KW), 1)

    def body(t, carry):
        mall = m_ref[...]
        m = jnp.min(mall, axis=1, keepdims=True)
        ci = jnp.min(jnp.where(mall == m, cio, _KC), axis=1, keepdims=True)
        for c in range(_KC):
            cs = c * _KW
            hit = jnp.logical_and(active[c], jnp.any(ci == c))

            @pl.when(hit)
            def _(c=c, cs=cs):
                rows = ci == c
                dch = d_ref[:, cs:cs + _KW]
                amin = jnp.min(
                    jnp.where(jnp.logical_and(rows, dch == m), wio, _KW),
                    axis=1, keepdims=True)
                upd = amin < _KW
                newch = jnp.where(wio == amin, jnp.inf, dch)
                d_ref[:, cs:cs + _KW] = newch
                m_ref[:, c:c + 1] = jnp.where(
                    upd, jnp.min(newch, axis=1, keepdims=True),
                    m_ref[:, c:c + 1])
                idx_ref[...] = jnp.where(
                    jnp.logical_and(upd, kio == t), cs + amin, idx_ref[...])
        return carry

    lax.fori_loop(0, KNN, body, 0)


def _knn(x, batch, bcol, brow):
    n, c = x.shape
    return pl.pallas_call(
        _knn_body,
        grid=(n // _KR,),
        in_specs=[
            pl.BlockSpec(memory_space=pltpu.SMEM),
            pl.BlockSpec((_KR, c), lambda i: (i, 0)),
            pl.BlockSpec((c, n), lambda i: (0, 0)),
            pl.BlockSpec((_KR, 1), lambda i: (i, 0)),
            pl.BlockSpec((1, n), lambda i: (0, 0)),
        ],
        out_specs=pl.BlockSpec((_KR, KNN), lambda i: (i, 0)),
        out_shape=jax.ShapeDtypeStruct((n, KNN), jnp.int32),
        scratch_shapes=[pltpu.VMEM((_KR, NPTS), jnp.float32),
                        pltpu.VMEM((_KR, _KC), jnp.float32)],
    )(batch.reshape(1, n), x, x.T, bcol, brow)


def _gather_rows(table, idx):
    nrow, d = table.shape
    b = idx.shape[0]
    nw = 32
    bw = b // nw
    chunk = 512 if d > 64 else 1024
    nch = bw // chunk
    mesh = plsc.VectorSubcoreMesh(core_axis_name="c", subcore_axis_name="s")

    @functools.partial(
        pl.kernel,
        out_type=jax.ShapeDtypeStruct((b, d), jnp.float32),
        mesh=mesh,
        scratch_types=[
            pltpu.VMEM((chunk,), jnp.int32),
            pltpu.VMEM((chunk, d), jnp.float32),
            pltpu.SemaphoreType.DMA,
        ],
    )
    def gk(table_hbm, idx_hbm, out_hbm, idx_v, rows_v, sem):
        wid = lax.axis_index("s") * 2 + lax.axis_index("c")

        def body(j, carry):
            base = wid * bw + j * chunk
            pltpu.sync_copy(idx_hbm.at[pl.ds(base, chunk)], idx_v)
            pltpu.async_copy(table_hbm.at[idx_v], rows_v, sem).wait()
            pltpu.sync_copy(rows_v, out_hbm.at[pl.ds(base, chunk)])
            return carry

        lax.fori_loop(0, nch, body, 0)

    return gk(table, idx)


def _edge1_body(xj_ref, xi_ref, w_ref, b_ref, h_ref):
    xi = xi_ref[...]
    el = xj_ref[0][:, :8] - xi
    e = jnp.concatenate([xi[:, :3], el[:, :5]], axis=1)
    h_ref[0] = _dot(e, w_ref[...])


def _edge1(xj3, xi, w, brow):
    bp = 2048
    cw = xj3.shape[2]
    cout = w.shape[1]
    return pl.pallas_call(
        _edge1_body,
        grid=(KNN, NPTS // bp),
        in_specs=[
            pl.BlockSpec((1, bp, cw), lambda k, i: (k, i, 0)),
            pl.BlockSpec((bp, xi.shape[1]), lambda k, i: (i, 0)),
            pl.BlockSpec(w.shape, lambda k, i: (0, 0)),
            pl.BlockSpec((1, cout), lambda k, i: (0, 0)),
        ],
        out_specs=pl.BlockSpec((1, bp, cout), lambda k, i: (k, i, 0)),
        out_shape=jax.ShapeDtypeStruct((KNN, NPTS, cout), jnp.float32),
    )(xj3, xi, w, brow)


def _bn(x, g, e, m, den):
    return g * (x - m) / den + e


def _mlpaff_body(x_ref, pb_ref, g_ref, e_ref, m_ref, dn_ref, w_ref, h_ref):
    h = _lrelu(x_ref[...] + pb_ref[...])
    z = _bn(h, g_ref[...], e_ref[...], m_ref[...], dn_ref[...])
    h_ref[...] = _dot(z, w_ref[...])


def _mlpaff(x, pbrow, g, e, mm, den, w, blk=4096):
    m, cin = x.shape
    cout = w.shape[1]
    return pl.pallas_call(
        _mlpaff_body,
        grid=(m // blk,),
        in_specs=[
            pl.BlockSpec((blk, cin), lambda i: (i, 0)),
            pl.BlockSpec((1, cin), lambda i: (0, 0)),
            pl.BlockSpec((1, cin), lambda i: (0, 0)),
            pl.BlockSpec((1, cin), lambda i: (0, 0)),
            pl.BlockSpec((1, cin), lambda i: (0, 0)),
            pl.BlockSpec((1, cin), lambda i: (0, 0)),
            pl.BlockSpec((cin, cout), lambda i: (0, 0)),
        ],
        out_specs=pl.BlockSpec((blk, cout), lambda i: (i, 0)),
        out_shape=jax.ShapeDtypeStruct((m, cout), jnp.float32),
    )(x, pbrow, g, e, mm, den, w)


def _edge2_body(xj_ref, xi_ref, w_ref, b_ref, h_ref):
    ci = xi_ref.shape[1]
    xi = xi_ref[...]
    el = xj_ref[0][:, :ci] - xi
    e = jnp.concatenate([xi, el], axis=1)
    h_ref[0] = _dot(e, w_ref[...])


def _edge2(xj3, xi, w, brow):
    bp = 1024
    cw = xj3.shape[2]
    cout = w.shape[1]
    return pl.pallas_call(
        _edge2_body,
        grid=(KNN, NPTS // bp),
        in_specs=[
            pl.BlockSpec((1, bp, cw), lambda k, i: (k, i, 0)),
            pl.BlockSpec((bp, xi.shape[1]), lambda k, i: (i, 0)),
            pl.BlockSpec(w.shape, lambda k, i: (0, 0)),
            pl.BlockSpec((1, cout), lambda k, i: (0, 0)),
        ],
        out_specs=pl.BlockSpec((1, bp, cout), lambda k, i: (k, i, 0)),
        out_shape=jax.ShapeDtypeStruct((KNN, NPTS, cout), jnp.float32),
    )(xj3, xi, w, brow)


def _bnmax_body(h_ref, pb_ref, g_ref, e_ref, m_ref, dn_ref, mx_ref):
    h = _lrelu(h_ref[0] + pb_ref[...])
    z = _bn(h, g_ref[...], e_ref[...], m_ref[...], dn_ref[...])
    k = pl.program_id(1)

    @pl.when(k == 0)
    def _():
        mx_ref[...] = z

    @pl.when(k > 0)
    def _():
        mx_ref[...] = jnp.maximum(mx_ref[...], z)


def _bnmax(hkm, pbrow, g, e, mm, den):
    _, n, c = hkm.shape
    bp = 1024
    return pl.pallas_call(
        _bnmax_body,
        grid=(n // bp, KNN),
        in_specs=[
            pl.BlockSpec((1, bp, c), lambda i, k: (k, i, 0)),
            pl.BlockSpec((1, c), lambda i, k: (0, 0)),
            pl.BlockSpec((1, c), lambda i, k: (0, 0)),
            pl.BlockSpec((1, c), lambda i, k: (0, 0)),
            pl.BlockSpec((1, c), lambda i, k: (0, 0)),
            pl.BlockSpec((1, c), lambda i, k: (0, 0)),
        ],
        out_specs=pl.BlockSpec((bp, c), lambda i, k: (i, 0)),
        out_shape=jax.ShapeDtypeStruct((n, c), jnp.float32),
    )(hkm, pbrow, g, e, mm, den)


def _l1_body(x1_ref, x2_ref, w_ref, b_ref, bt_ref, emb_ref, st_ref):
    cat = jnp.concatenate([x1_ref[...], x2_ref[...]], axis=1)
    h = _lrelu(_dot(cat, w_ref[...]) + b_ref[...])

    @pl.when(pl.program_id(0) == 0)
    def _():
        emb_ref[...] = jnp.full_like(emb_ref, _NEG)
        st_ref[...] = jnp.zeros_like(st_ref)

    st_ref[...] += jnp.stack(
        [jnp.sum(h, axis=0), jnp.sum(h * h, axis=0)])

    bt = bt_ref[...]
    for g in range(NGR):
        red = jnp.max(jnp.where(bt == g, h, _NEG), axis=0, keepdims=True)
        emb_ref[pl.ds(g, 1), :] = jnp.maximum(emb_ref[pl.ds(g, 1), :], red)


def _l1_segmax(x1, x2, w, brow, bcol):
    n = x1.shape[0]
    cout = w.shape[1]
    bp = 1024
    return pl.pallas_call(
        _l1_body,
        grid=(n // bp,),
        in_specs=[
            pl.BlockSpec((bp, x1.shape[1]), lambda i: (i, 0)),
            pl.BlockSpec((bp, x2.shape[1]), lambda i: (i, 0)),
            pl.BlockSpec(w.shape, lambda i: (0, 0)),
            pl.BlockSpec((1, cout), lambda i: (0, 0)),
            pl.BlockSpec((bp, 1), lambda i: (i, 0)),
        ],
        out_specs=[
            pl.BlockSpec((NGR, cout), lambda i: (0, 0)),
            pl.BlockSpec((2, cout), lambda i: (0, 0)),
        ],
        out_shape=[
            jax.ShapeDtypeStruct((NGR, cout), jnp.float32),
            jax.ShapeDtypeStruct((2, cout), jnp.float32),
        ],
    )(x1, x2, w, brow, bcol)


def _head_body(ep_ref, g5_ref, e5_ref, m5_ref, d5_ref,
               w1_ref, b1_ref, g1_ref, e1_ref,
               w2_ref, b2_ref, g2_ref, e2_ref, wf_ref, bf_ref,
               emb_ref, log_ref):
    emb = _bn(ep_ref[...], g5_ref[...], e5_ref[...], m5_ref[...], d5_ref[...])
    emb_ref[...] = emb

    def layer(x, w, b, g, e):
        h = _lrelu(_dot(x, w[...]) + b[...])
        m = jnp.mean(h, axis=0, keepdims=True)
        v = jnp.mean((h - m) * (h - m), axis=0, keepdims=True)
        return g[...] * (h - m) / jnp.sqrt(v + 1e-5) + e[...]

    h = layer(emb, w1_ref, b1_ref, g1_ref, e1_ref)
    h = layer(h, w2_ref, b2_ref, g2_ref, e2_ref)
    log_ref[...] = _dot(h, wf_ref[...]) + bf_ref[...]


def _head(embpre, g5, e5, m5, d5, w1, b1, g1, e1, w2, b2, g2, e2, wf, bf):
    return pl.pallas_call(
        _head_body,
        out_shape=[
            jax.ShapeDtypeStruct((NGR, embpre.shape[1]), jnp.float32),
            jax.ShapeDtypeStruct((NGR, wf.shape[1]), jnp.float32),
        ],
    )(embpre, g5, e5, m5, d5, w1, b1, g1, e1, w2, b2, g2, e2, wf, bf)


def _bn_params(stats, cnt):
    s, s2 = stats[0], stats[1]
    m = s / cnt
    v = s2 / cnt - m * m
    return m.reshape(1, -1), jnp.sqrt(v + 1e-5).reshape(1, -1)


def _bn_params_of(tkm, b):
    c = tkm.shape[2]
    hn = tkm.reshape(NPTS * KNN, c) + b
    hn = jnp.where(hn >= 0, hn, 0.01 * hn)
    m = jnp.mean(hn, axis=0)
    v = jnp.var(hn, axis=0)
    return m.reshape(1, -1), jnp.sqrt(v + 1e-5).reshape(1, -1)


def _row(v):
    return v.reshape(1, -1)


def kernel(pos, batch, c1_W0, c1_b0, c1_g0, c1_e0, c1_W1, c1_b1, c1_g1,
           c1_e1, c1_W2, c1_b2, c1_g2, c1_e2, c2_W0, c2_b0, c2_g0, c2_e0,
           l1_W0, l1_b0, l1_g0, l1_e0, h1_W0, h1_b0, h1_g0, h1_e0,
           h2_W0, h2_b0, h2_g0, h2_e0, fc_W, fc_b):
    ne = NPTS * KNN
    bcol = batch.reshape(NPTS, 1)
    brow = batch.reshape(1, NPTS)

    posp = jnp.pad(pos, ((0, 0), (0, 5)))
    posw = jnp.pad(pos, ((0, 0), (0, 125)))
    idx1 = _knn(posp, batch, bcol, brow)
    idx1f = idx1.T.reshape(ne)

    xj1 = _gather_rows(posw, idx1f)
    w8 = jnp.pad(c1_W0, ((0, 2), (0, 0)))
    t1f = _edge1(xj1.reshape(KNN, NPTS, 128), posp, w8, _row(c1_b0))

    m1, d1 = _bn_params_of(t1f, c1_b0)
    t2f = _mlpaff(t1f.reshape(ne, 64), _row(c1_b0), _row(c1_g0),
                  _row(c1_e0), m1, d1, c1_W1)

    m2, d2 = _bn_params_of(t2f.reshape(KNN, NPTS, 64), c1_b1)
    t3f = _mlpaff(t2f, _row(c1_b1), _row(c1_g1), _row(c1_e1), m2, d2,
                  c1_W2).reshape(KNN, NPTS, 64)

    m3, d3 = _bn_params_of(t3f, c1_b2)
    x1 = _bnmax(t3f, _row(c1_b2), _row(c1_g2), _row(c1_e2), m3, d3)

    idx2 = _knn(x1, batch, bcol, brow)
    idx2f = idx2.T.reshape(ne)

    x1w = jnp.pad(x1, ((0, 0), (0, 64)))
    xj2 = _gather_rows(x1w, idx2f)
    t4f = _edge2(xj2.reshape(KNN, NPTS, 128), x1, c2_W0, _row(c2_b0))

    m4, d4 = _bn_params_of(t4f, c2_b0)
    x2 = _bnmax(t4f, _row(c2_b0), _row(c2_g0), _row(c2_e0), m4, d4)

    embpre, st5 = _l1_segmax(x1, x2, l1_W0, _row(l1_b0), bcol)

    m5, d5 = _bn_params(st5, float(NPTS))

    emb, logits = _head(
        embpre, _row(l1_g0), _row(l1_e0), m5, d5,
        h1_W0, _row(h1_b0), _row(h1_g0), _row(h1_e0),
        h2_W0, _row(h2_b0), _row(h2_g0), _row(h2_e0),
        fc_W, _row(fc_b))
    return (logits, emb)
